# baseline (speedup 1.0000x reference)
import jax
import jax.numpy as jnp
from jax import lax
from jax.experimental import pallas as pl
from jax.experimental.pallas import tpu as pltpu

ROWS = 128
COLS = 100000
SUB = 64
BLK_A = 2048
BLK_W = 2000


def _threefry_bits(x1i):
    ks0 = jnp.uint32(0)
    ks1 = jnp.uint32(42)
    ks2 = ks0 ^ ks1 ^ jnp.uint32(0x1BD11BDA)
    ks = (ks0, ks1, ks2)
    rots = ((13, 15, 26, 6), (17, 29, 16, 24))

    def rotl(v, r):
        return (v << jnp.uint32(r)) | (v >> jnp.uint32(32 - r))

    x0 = x1i
    x1 = x0 ^ rotl(x1i, rots[0][0])
    for r in rots[0][1:]:
        x0 = x0 + x1
        x1 = x0 ^ rotl(x1, r)
    x0 = x0 + ks[1]
    x1 = x1 + ks[2] + jnp.uint32(1)
    for i in range(1, 5):
        for r in rots[i % 2]:
            x0 = x0 + x1
            x1 = x0 ^ rotl(x1, r)
        x0 = x0 + ks[(i + 1) % 3]
        x1 = x1 + ks[(i + 2) % 3] + jnp.uint32(i + 1)
    return x0 ^ x1


def _gumbel(x1i):
    bits = _threefry_bits(x1i)
    fb = (bits >> jnp.uint32(9)) | jnp.uint32(0x3F800000)
    f = lax.bitcast_convert_type(fb, jnp.float32) - jnp.float32(1.0)
    u = jnp.maximum(f, jnp.float32(1e-20))
    return -jnp.log(-jnp.log(u))


def _argmax_kernel(x_ref, idx_ref, val_s, col_s):
    j = pl.program_id(0)
    nb = pl.num_programs(0)

    @pl.when(j == 0)
    def _():
        val_s[...] = jnp.full((SUB, ROWS), -jnp.inf, jnp.float32)
        col_s[...] = jnp.zeros((SUB, ROWS), jnp.int32)

    lanes_r = lax.broadcasted_iota(jnp.uint32, (SUB, ROWS), 1)
    subi = lax.broadcasted_iota(jnp.int32, (SUB, ROWS), 0)
    q = lanes_r * jnp.uint32(COLS) + subi.astype(jnp.uint32)

    acc_v = val_s[...]
    acc_c = col_s[...]
    for k in range(BLK_A // SUB):
        xs = x_ref[k * SUB:(k + 1) * SUB, :]
        c0 = j * BLK_A + k * SUB
        ccol = subi + c0
        x1i = q + (c0.astype(jnp.uint32) + jnp.uint32(42))
        chunk = xs + _gumbel(x1i)
        chunk = jnp.where(ccol < COLS, chunk, -jnp.inf)
        better = chunk > acc_v
        acc_v = jnp.where(better, chunk, acc_v)
        acc_c = jnp.where(better, ccol, acc_c)
    val_s[...] = acc_v
    col_s[...] = acc_c

    @pl.when(j == nb - 1)
    def _():
        m = jnp.max(acc_v, axis=0, keepdims=True)
        cand = jnp.where(acc_v == m, acc_c, jnp.int32(2**31 - 1))
        idx_ref[...] = jnp.min(cand, axis=0, keepdims=True)


def _onehot_kernel(idx_ref, o_ref):
    j = pl.program_id(0)
    rowid = lax.broadcasted_iota(jnp.int32, (BLK_W, ROWS), 0) + j * BLK_W
    o_ref[...] = (rowid == idx_ref[...]).astype(jnp.float32)


def kernel(x):
    xt = x.T
    idx = pl.pallas_call(
        _argmax_kernel,
        grid=(pl.cdiv(COLS, BLK_A),),
        in_specs=[pl.BlockSpec((BLK_A, ROWS), lambda j: (j, 0))],
        out_specs=pl.BlockSpec((1, ROWS), lambda j: (0, 0)),
        out_shape=jax.ShapeDtypeStruct((1, ROWS), jnp.int32),
        scratch_shapes=[
            pltpu.VMEM((SUB, ROWS), jnp.float32),
            pltpu.VMEM((SUB, ROWS), jnp.int32),
        ],
    )(xt)
    out_t = pl.pallas_call(
        _onehot_kernel,
        grid=(pl.cdiv(COLS, BLK_W),),
        in_specs=[pl.BlockSpec((1, ROWS), lambda j: (0, 0))],
        out_specs=pl.BlockSpec((BLK_W, ROWS), lambda j: (j, 0)),
        out_shape=jax.ShapeDtypeStruct((COLS, ROWS), jnp.float32),
    )(idx)
    return out_t.T


# speedup vs baseline: 1.0424x; 1.0424x over previous
import functools

import jax
import jax.numpy as jnp
from jax import lax
from jax.experimental import pallas as pl
from jax.experimental.pallas import tpu as pltpu
from jax.experimental.pallas import tpu_sc as plsc

ROWS = 128
COLS = 100000
SUB = 64
BLK_A = 2048
NFLAT = ROWS * COLS
SC_WORKERS = 32
SC_PER_W = NFLAT // SC_WORKERS
SC_ZBUF = 8192


def _threefry_bits(p):
    ks0 = jnp.uint32(0)
    ks1 = jnp.uint32(42)
    ks2 = ks0 ^ ks1 ^ jnp.uint32(0x1BD11BDA)
    ks = (ks0, ks1, ks2)
    rots = ((13, 15, 26, 6), (17, 29, 16, 24))

    def rotl(v, r):
        return (v << jnp.uint32(r)) | (v >> jnp.uint32(32 - r))

    x1i = p + ks1
    x0 = x1i
    x1 = x0 ^ rotl(x1i, rots[0][0])
    for r in rots[0][1:]:
        x0 = x0 + x1
        x1 = x0 ^ rotl(x1, r)
    x0 = x0 + ks[1]
    x1 = x1 + ks[2] + jnp.uint32(1)
    for i in range(1, 5):
        for r in rots[i % 2]:
            x0 = x0 + x1
            x1 = x0 ^ rotl(x1, r)
        x0 = x0 + ks[(i + 1) % 3]
        x1 = x1 + ks[(i + 2) % 3] + jnp.uint32(i + 1)
    return x0 ^ x1


def _gumbel(p):
    bits = _threefry_bits(p)
    fb = (bits >> jnp.uint32(9)) | jnp.uint32(0x3F800000)
    f = lax.bitcast_convert_type(fb, jnp.float32) - jnp.float32(1.0)
    u = jnp.maximum(f, jnp.float32(1e-20))
    return -jnp.log(-jnp.log(u))


def _argmax_kernel(x_ref, idx_ref, val_s, col_s):
    j = pl.program_id(0)
    nb = pl.num_programs(0)

    @pl.when(j == 0)
    def _():
        val_s[...] = jnp.full((SUB, ROWS), -jnp.inf, jnp.float32)
        col_s[...] = jnp.zeros((SUB, ROWS), jnp.int32)

    lanes_r = lax.broadcasted_iota(jnp.uint32, (SUB, ROWS), 1)
    subi = lax.broadcasted_iota(jnp.int32, (SUB, ROWS), 0)
    rbase = lanes_r * jnp.uint32(COLS)

    acc_v = val_s[...]
    acc_c = col_s[...]
    for k in range(BLK_A // SUB):
        xs = x_ref[k * SUB:(k + 1) * SUB, :]
        ccol = subi + (j * BLK_A + k * SUB)
        p = rbase + ccol.astype(jnp.uint32)
        chunk = xs + _gumbel(p)
        chunk = jnp.where(ccol < COLS, chunk, -jnp.inf)
        better = chunk > acc_v
        acc_v = jnp.where(better, chunk, acc_v)
        acc_c = jnp.where(better, ccol, acc_c)
    val_s[...] = acc_v
    col_s[...] = acc_c

    @pl.when(j == nb - 1)
    def _():
        m = jnp.max(acc_v, axis=0, keepdims=True)
        cand = jnp.where(acc_v == m, acc_c, jnp.int32(2**31 - 1))
        idx_ref[...] = jnp.min(cand, axis=0, keepdims=True)


def _sc_zeros():
    mesh = plsc.VectorSubcoreMesh(core_axis_name="c", subcore_axis_name="s")

    @functools.partial(
        pl.kernel,
        mesh=mesh,
        out_type=jax.ShapeDtypeStruct((NFLAT,), jnp.float32),
        scratch_types=[pltpu.VMEM((SC_ZBUF,), jnp.float32)],
    )
    def zero_kernel(out_hbm, zbuf):
        wid = lax.axis_index("s") * 2 + lax.axis_index("c")
        zv = jnp.zeros((16,), jnp.float32)

        def zinit(i, _):
            zbuf[pl.ds(i * 16, 16)] = zv
            return ()

        lax.fori_loop(0, SC_ZBUF // 16, zinit, ())
        base = wid * SC_PER_W

        def body(t, _):
            pltpu.sync_copy(zbuf, out_hbm.at[pl.ds(base + t * SC_ZBUF, SC_ZBUF)])
            return ()

        nfull = SC_PER_W // SC_ZBUF
        lax.fori_loop(0, nfull, body, ())
        rem = SC_PER_W - nfull * SC_ZBUF
        if rem:
            pltpu.sync_copy(
                zbuf.at[pl.ds(0, rem)],
                out_hbm.at[pl.ds(base + nfull * SC_ZBUF, rem)],
            )

    return zero_kernel()


def _scatter_kernel(idx_smem, zeros_ref, idx_vmem, out_ref, m_vmem, sem):
    del zeros_ref
    ida = idx_vmem[...]
    for r in range(ROWS):
        s = idx_smem[0, r]
        row = (ida == s).astype(jnp.float32)
        m_vmem[r, :] = row[0, :]
    for r in range(ROWS):
        s = idx_smem[0, r]
        pltpu.make_async_copy(
            m_vmem.at[pl.ds(r, 1), :], out_ref.at[pl.ds(s, 1), :], sem
        ).start()
    for r in range(ROWS):
        s = idx_smem[0, r]
        pltpu.make_async_copy(
            m_vmem.at[pl.ds(r, 1), :], out_ref.at[pl.ds(s, 1), :], sem
        ).wait()


def _tc_scatter(zeros, idx):
    return pl.pallas_call(
        _scatter_kernel,
        in_specs=[
            pl.BlockSpec(memory_space=pltpu.SMEM),
            pl.BlockSpec(memory_space=pl.ANY),
            pl.BlockSpec(memory_space=pltpu.VMEM),
        ],
        out_specs=pl.BlockSpec(memory_space=pl.ANY),
        out_shape=jax.ShapeDtypeStruct((COLS, ROWS), jnp.float32),
        input_output_aliases={1: 0},
        scratch_shapes=[
            pltpu.VMEM((ROWS, ROWS), jnp.float32),
            pltpu.SemaphoreType.DMA,
        ],
    )(idx, zeros, idx)


def kernel(x):
    xt = x.T
    zeros = _sc_zeros().reshape(COLS, ROWS)
    idx = pl.pallas_call(
        _argmax_kernel,
        grid=(pl.cdiv(COLS, BLK_A),),
        in_specs=[pl.BlockSpec((BLK_A, ROWS), lambda j: (j, 0))],
        out_specs=pl.BlockSpec((1, ROWS), lambda j: (0, 0)),
        out_shape=jax.ShapeDtypeStruct((1, ROWS), jnp.int32),
        scratch_shapes=[
            pltpu.VMEM((SUB, ROWS), jnp.float32),
            pltpu.VMEM((SUB, ROWS), jnp.int32),
        ],
    )(xt)
    out_t = _tc_scatter(zeros, idx)
    return out_t.T
